# Initial kernel scaffold; baseline (speedup 1.0000x reference)
import jax
import jax.numpy as jnp
from jax.experimental import pallas as pl


def kernel(x, pos_edge_index, W1, b1, W2, b2):
    raise NotImplementedError("write your pallas kernel here")


# speedup vs baseline: 4.0725x; 4.0725x over previous
import functools

import jax
import jax.numpy as jnp
from jax import lax
from jax.experimental import pallas as pl
from jax.experimental.pallas import tpu as pltpu
from jax.experimental.pallas import tpu_sc as plsc

N = 10000
E = 160000
NC = 2
NS = 16
L = 16
NW = NC * NS

BATCH = 128
E_PAD = 163840
ROWS_ALL = E_PAD // BATCH
N_PAD = 10240
NT = N_PAD // NS

D_PAD = 640
NCHUNK = 8
CW = D_PAD // NCHUNK
CROWS = N + 1
D2 = 64

@functools.lru_cache(maxsize=None)
def _mesh():
    return plsc.VectorSubcoreMesh(core_axis_name="c", subcore_axis_name="s",
                                  num_cores=NC, num_subcores=NS)


_SC_PARAMS = pltpu.CompilerParams(needs_layout_passes=False,
                                  use_tc_tiling_on_sc=False)


def _zero_vmem(ref, nrows, ncols):
    def row(i, _):
        for k in range(ncols // L):
            ref[i, pl.ds(k * L, L)] = jnp.zeros((L,), jnp.float32)
        return 0
    lax.fori_loop(0, nrows, row, 0)


def _deg_body(dst2d_hbm, degp_hbm, dstv, deg_v, tmp_v, stage_sh):
    cid = lax.axis_index("c")
    sid = lax.axis_index("s")
    wid = cid * NS + sid
    rows_per_tile = ROWS_ALL // NW

    def z(i, _):
        deg_v[pl.ds(i * L, L)] = jnp.zeros((L,), jnp.float32)
        return 0
    lax.fori_loop(0, N_PAD // L, z, 0)

    pltpu.sync_copy(dst2d_hbm.at[pl.ds(wid * rows_per_tile, rows_per_tile)],
                    dstv)

    ones = jnp.ones((L,), jnp.float32)

    def scat(j, _):
        for k in range(BATCH // L):
            idx = dstv[j, pl.ds(k * L, L)]
            plsc.addupdate_scatter(deg_v, [idx], ones)
        return 0
    lax.fori_loop(0, rows_per_tile, scat, 0)

    pltpu.sync_copy(deg_v, stage_sh.at[sid])
    plsc.subcore_barrier()

    def zz(i, _):
        deg_v[pl.ds(i * L, L)] = jnp.zeros((L,), jnp.float32)
        return 0
    lax.fori_loop(0, NT // L, zz, 0)

    def red(k, _):
        pltpu.sync_copy(stage_sh.at[k, pl.ds(sid * NT, NT)], tmp_v)
        for v in range(NT // L):
            s = pl.ds(v * L, L)
            deg_v[s] = deg_v[s] + tmp_v[s]
        return 0
    lax.fori_loop(0, NS, red, 0)

    pltpu.sync_copy(deg_v.at[pl.ds(0, NT)],
                    degp_hbm.at[cid, pl.ds(sid * NT, NT)])


@functools.lru_cache(maxsize=None)
def _deg_kernel():
    return pl.kernel(
        _deg_body,
        out_type=jax.ShapeDtypeStruct((NC, N_PAD), jnp.float32),
        mesh=_mesh(),
        compiler_params=_SC_PARAMS,
        scratch_types=[
            pltpu.VMEM((ROWS_ALL // NW, BATCH), jnp.int32),
            pltpu.VMEM((N_PAD,), jnp.float32),
            pltpu.VMEM((NT,), jnp.float32),
            pltpu.VMEM_SHARED((NS, N_PAD), jnp.float32),
        ],
    )


def _agg1_body(xs_cat_hbm, src2d_hbm, dst2d_hbm, out_hbm,
               srcv, dstv, gidx, rowbuf, zbuf, acc_sh):
    cid = lax.axis_index("c")
    sid = lax.axis_index("s")
    rows_per_tile = ROWS_ALL // NS

    pltpu.sync_copy(src2d_hbm.at[pl.ds(sid * rows_per_tile, rows_per_tile)],
                    srcv)
    pltpu.sync_copy(dst2d_hbm.at[pl.ds(sid * rows_per_tile, rows_per_tile)],
                    dstv)

    _zero_vmem(zbuf, BATCH, CW)

    for ci in range(NCHUNK // NC):
        c = cid * (NCHUNK // NC) + ci
        base = c * CROWS

        def gx(j, _):
            for k in range(BATCH // L):
                s = pl.ds(k * L, L)
                gidx[j, s] = srcv[j, s] + base
            return 0
        lax.fori_loop(0, rows_per_tile, gx, 0)

        def zacc(i, _):
            pltpu.sync_copy(zbuf, acc_sh.at[pl.ds(sid * NT + i * BATCH,
                                                  BATCH)])
            return 0
        lax.fori_loop(0, NT // BATCH, zacc, 0)
        plsc.subcore_barrier()

        def step(j, _):
            pltpu.sync_copy(xs_cat_hbm.at[gidx.at[j]], rowbuf)
            pltpu.sync_copy(rowbuf, acc_sh.at[dstv.at[j]], add=True)
            return 0
        lax.fori_loop(0, rows_per_tile, step, 0)
        plsc.subcore_barrier()

        def flush(i, _):
            r0 = sid * NT + i * BATCH
            pltpu.sync_copy(acc_sh.at[pl.ds(r0, BATCH)], rowbuf)
            pltpu.sync_copy(rowbuf, out_hbm.at[c, pl.ds(r0, BATCH)])
            return 0
        lax.fori_loop(0, NT // BATCH, flush, 0)
        plsc.subcore_barrier()


@functools.lru_cache(maxsize=None)
def _agg1_kernel():
    return pl.kernel(
        _agg1_body,
        out_type=jax.ShapeDtypeStruct((NCHUNK, N_PAD, CW), jnp.float32),
        mesh=_mesh(),
        compiler_params=_SC_PARAMS,
        scratch_types=[
            pltpu.VMEM((ROWS_ALL // NS, BATCH), jnp.int32),
            pltpu.VMEM((ROWS_ALL // NS, BATCH), jnp.int32),
            pltpu.VMEM((ROWS_ALL // NS, BATCH), jnp.int32),
            pltpu.VMEM((BATCH, CW), jnp.float32),
            pltpu.VMEM((BATCH, CW), jnp.float32),
            pltpu.VMEM_SHARED((N_PAD, CW), jnp.float32),
        ],
    )


def _agg2_body(gs_hbm, src2d_hbm, dst2d_hbm, out_hbm,
               srcv, dstv, rowbuf, zbuf, acc_sh):
    cid = lax.axis_index("c")
    sid = lax.axis_index("s")
    wid = cid * NS + sid
    rows_per_tile = ROWS_ALL // NW

    pltpu.sync_copy(src2d_hbm.at[pl.ds(wid * rows_per_tile, rows_per_tile)],
                    srcv)
    pltpu.sync_copy(dst2d_hbm.at[pl.ds(wid * rows_per_tile, rows_per_tile)],
                    dstv)

    _zero_vmem(zbuf, BATCH, D2)

    def zacc(i, _):
        pltpu.sync_copy(zbuf, acc_sh.at[pl.ds(sid * NT + i * BATCH, BATCH)])
        return 0
    lax.fori_loop(0, NT // BATCH, zacc, 0)
    plsc.subcore_barrier()

    def step(j, _):
        pltpu.sync_copy(gs_hbm.at[srcv.at[j]], rowbuf)
        pltpu.sync_copy(rowbuf, acc_sh.at[dstv.at[j]], add=True)
        return 0
    lax.fori_loop(0, rows_per_tile, step, 0)
    plsc.subcore_barrier()

    def flush(i, _):
        r0 = sid * NT + i * BATCH
        pltpu.sync_copy(acc_sh.at[pl.ds(r0, BATCH)], rowbuf)
        pltpu.sync_copy(rowbuf, out_hbm.at[cid, pl.ds(r0, BATCH)])
        return 0
    lax.fori_loop(0, NT // BATCH, flush, 0)


@functools.lru_cache(maxsize=None)
def _agg2_kernel():
    return pl.kernel(
        _agg2_body,
        out_type=jax.ShapeDtypeStruct((NC, N_PAD, D2), jnp.float32),
        mesh=_mesh(),
        compiler_params=_SC_PARAMS,
        scratch_types=[
            pltpu.VMEM((ROWS_ALL // NW, BATCH), jnp.int32),
            pltpu.VMEM((ROWS_ALL // NW, BATCH), jnp.int32),
            pltpu.VMEM((BATCH, D2), jnp.float32),
            pltpu.VMEM((BATCH, D2), jnp.float32),
            pltpu.VMEM_SHARED((N_PAD, D2), jnp.float32),
        ],
    )


def _pre_body(d0_ref, d1_ref, x_ref, xs_ref, dinv_ref):
    dcol = lax.rsqrt(d0_ref[...] + d1_ref[...] + 1.0)
    dinv_ref[...] = dcol
    xs_ref[...] = x_ref[...] * dcol


def _pre(d0, d1, x):
    rb = 2000
    return pl.pallas_call(
        _pre_body,
        grid=(N // rb,),
        in_specs=[
            pl.BlockSpec((rb, 1), lambda i: (i, 0)),
            pl.BlockSpec((rb, 1), lambda i: (i, 0)),
            pl.BlockSpec((rb, 600), lambda i: (i, 0)),
        ],
        out_specs=[
            pl.BlockSpec((rb, 600), lambda i: (i, 0)),
            pl.BlockSpec((rb, 1), lambda i: (i, 0)),
        ],
        out_shape=[
            jax.ShapeDtypeStruct((N, 600), jnp.float32),
            jax.ShapeDtypeStruct((N, 1), jnp.float32),
        ],
    )(d0, d1, x)


def _mid_body(agg_ref, xs_ref, dinv_ref, w1_ref, b1_ref, w2_ref, gs_ref):
    dcol = dinv_ref[...]
    u = (agg_ref[...] + xs_ref[...]) * dcol
    h = jax.lax.dot_general(u, w1_ref[...], (((1,), (1,)), ((), ())),
                            preferred_element_type=jnp.float32)
    h = jnp.maximum(h + b1_ref[...], 0.0)
    g = jax.lax.dot_general(h, w2_ref[...], (((1,), (1,)), ((), ())),
                            preferred_element_type=jnp.float32)
    gs_ref[...] = g * dcol


def _mid(agg, xs, dinv, w1, b1, w2):
    rb = 2000
    return pl.pallas_call(
        _mid_body,
        grid=(N // rb,),
        in_specs=[
            pl.BlockSpec((rb, 600), lambda i: (i, 0)),
            pl.BlockSpec((rb, 600), lambda i: (i, 0)),
            pl.BlockSpec((rb, 1), lambda i: (i, 0)),
            pl.BlockSpec((628, 600), lambda i: (0, 0)),
            pl.BlockSpec((628,), lambda i: (0,)),
            pl.BlockSpec((D2, 628), lambda i: (0, 0)),
        ],
        out_specs=pl.BlockSpec((rb, D2), lambda i: (i, 0)),
        out_shape=jax.ShapeDtypeStruct((N, D2), jnp.float32),
    )(agg, xs, dinv, w1, b1, w2)


def _fin_body(p0_ref, p1_ref, gs_ref, dinv_ref, b2_ref, z_ref):
    dcol = dinv_ref[...]
    z_ref[...] = (p0_ref[...] + p1_ref[...] + gs_ref[...]) * dcol + b2_ref[...]


def _fin(p0, p1, gs, dinv, b2):
    rb = 2000
    return pl.pallas_call(
        _fin_body,
        grid=(N // rb,),
        in_specs=[
            pl.BlockSpec((rb, D2), lambda i: (i, 0)),
            pl.BlockSpec((rb, D2), lambda i: (i, 0)),
            pl.BlockSpec((rb, D2), lambda i: (i, 0)),
            pl.BlockSpec((rb, 1), lambda i: (i, 0)),
            pl.BlockSpec((D2,), lambda i: (0,)),
        ],
        out_specs=pl.BlockSpec((rb, D2), lambda i: (i, 0)),
        out_shape=jax.ShapeDtypeStruct((N, D2), jnp.float32),
    )(p0, p1, gs, dinv, b2)


@jax.jit
def kernel(x, pos_edge_index, W1, b1, W2, b2):
    src = pos_edge_index[0].astype(jnp.int32)
    dst = pos_edge_index[1].astype(jnp.int32)
    pad = jnp.full((E_PAD - E,), N, jnp.int32)
    src2d = jnp.concatenate([src, pad]).reshape(ROWS_ALL, BATCH)
    dst2d = jnp.concatenate([dst, pad]).reshape(ROWS_ALL, BATCH)

    degp = _deg_kernel()(dst2d)
    d0 = degp[0, :N, None]
    d1 = degp[1, :N, None]
    xs, dinv = _pre(d0, d1, x)

    xs_pad = jnp.pad(xs, ((0, 1), (0, D_PAD - 600)))
    xs_cat = xs_pad.reshape(CROWS, NCHUNK, CW).transpose(1, 0, 2)
    xs_cat = xs_cat.reshape(NCHUNK * CROWS, CW)

    agg_chunks = _agg1_kernel()(xs_cat, src2d, dst2d)
    agg = agg_chunks[:, :N, :].transpose(1, 0, 2).reshape(N, D_PAD)[:, :600]

    gs = _mid(agg, xs, dinv, W1, b1, W2)
    gs_pad = jnp.pad(gs, ((0, 1), (0, 0)))

    parts = _agg2_kernel()(gs_pad, src2d, dst2d)
    z = _fin(parts[0, :N], parts[1, :N], gs, dinv, b2)
    return z


# speedup vs baseline: 6.9219x; 1.6997x over previous
import functools

import jax
import jax.numpy as jnp
from jax import lax
from jax.experimental import pallas as pl
from jax.experimental.pallas import tpu as pltpu
from jax.experimental.pallas import tpu_sc as plsc

N = 10000
E = 160000
NC = 2
NS = 16
L = 16
NW = NC * NS

BATCH = 128
E_PAD = 163840
ROWS_ALL = E_PAD // BATCH
N_PAD = 10240
NT = N_PAD // NS

D_PAD = 640
NCHUNK = 10
CW = D_PAD // NCHUNK
D2 = 64
NBUF = 4

@functools.lru_cache(maxsize=None)
def _mesh():
    return plsc.VectorSubcoreMesh(core_axis_name="c", subcore_axis_name="s",
                                  num_cores=NC, num_subcores=NS)


_SC_PARAMS = pltpu.CompilerParams(needs_layout_passes=False,
                                  use_tc_tiling_on_sc=False)


def _zero_vmem(ref, nrows, ncols):
    def row(i, _):
        for k in range(ncols // L):
            ref[i, pl.ds(k * L, L)] = jnp.zeros((L,), jnp.float32)
        return 0
    lax.fori_loop(0, nrows, row, 0)


def _deg_body(dst2d_hbm, degp_hbm, dstv, deg_v):
    cid = lax.axis_index("c")
    sid = lax.axis_index("s")
    wid = cid * NS + sid
    rows_per_tile = ROWS_ALL // NW

    def z(i, _):
        deg_v[pl.ds(i * L, L)] = jnp.zeros((L,), jnp.float32)
        return 0
    lax.fori_loop(0, N_PAD // L, z, 0)

    pltpu.sync_copy(dst2d_hbm.at[pl.ds(wid * rows_per_tile, rows_per_tile)],
                    dstv)

    ones = jnp.ones((L,), jnp.float32)

    def scat(j, _):
        for k in range(BATCH // L):
            idx = dstv[j, pl.ds(k * L, L)]
            plsc.addupdate_scatter(deg_v, [idx], ones)
        return 0
    lax.fori_loop(0, rows_per_tile, scat, 0)

    pltpu.sync_copy(deg_v, degp_hbm.at[wid])


@functools.lru_cache(maxsize=None)
def _deg_kernel():
    return pl.kernel(
        _deg_body,
        out_type=jax.ShapeDtypeStruct((NW, N_PAD), jnp.float32),
        mesh=_mesh(),
        compiler_params=_SC_PARAMS,
        scratch_types=[
            pltpu.VMEM((ROWS_ALL // NW, BATCH), jnp.int32),
            pltpu.VMEM((N_PAD,), jnp.float32),
        ],
    )


def _ring_agg(table_hbm, gidx, dstv, acc_sh, bufs, sems, nb):
    for b in range(NBUF):
        pltpu.async_copy(table_hbm.at[gidx.at[b]], bufs[b], sems[b])

    def outer(t, _):
        for b in range(NBUF):
            j = t * NBUF + b

            pltpu.make_async_copy(table_hbm.at[gidx.at[j]], bufs[b],
                                  sems[b]).wait()
            pltpu.sync_copy(bufs[b], acc_sh.at[dstv.at[j]], add=True)

            @pl.when(t < nb // NBUF - 1)
            def _():
                pltpu.async_copy(table_hbm.at[gidx.at[j + NBUF]], bufs[b],
                                 sems[b])
        return 0
    lax.fori_loop(0, nb // NBUF, outer, 0)


def _agg1_body(xs_cat_hbm, src2d_hbm, dst2d_hbm, out_hbm,
               srcv, dstv, gidx, b0, b1, b2, b3, zbuf, acc_sh,
               s0, s1, s2, s3):
    cid = lax.axis_index("c")
    sid = lax.axis_index("s")
    rows_per_tile = ROWS_ALL // NS
    bufs = [b0, b1, b2, b3]
    sems = [s0, s1, s2, s3]

    pltpu.sync_copy(src2d_hbm.at[pl.ds(sid * rows_per_tile, rows_per_tile)],
                    srcv)
    pltpu.sync_copy(dst2d_hbm.at[pl.ds(sid * rows_per_tile, rows_per_tile)],
                    dstv)

    _zero_vmem(zbuf, BATCH, CW)

    for ci in range(NCHUNK // NC):
        c = cid * (NCHUNK // NC) + ci
        base = c * N

        def gx(j, _):
            for k in range(BATCH // L):
                s = pl.ds(k * L, L)
                gidx[j, s] = srcv[j, s] + base
            return 0
        lax.fori_loop(0, rows_per_tile, gx, 0)

        def zacc(i, _):
            pltpu.sync_copy(zbuf, acc_sh.at[pl.ds(sid * NT + i * BATCH,
                                                  BATCH)])
            return 0
        lax.fori_loop(0, NT // BATCH, zacc, 0)
        plsc.subcore_barrier()

        _ring_agg(xs_cat_hbm, gidx, dstv, acc_sh, bufs, sems, rows_per_tile)
        plsc.subcore_barrier()

        def flush(i, _):
            r0 = sid * NT + i * BATCH
            pltpu.sync_copy(acc_sh.at[pl.ds(r0, BATCH)], b0)
            pltpu.sync_copy(b0, out_hbm.at[c, pl.ds(r0, BATCH)])
            return 0
        lax.fori_loop(0, NT // BATCH, flush, 0)
        plsc.subcore_barrier()


@functools.lru_cache(maxsize=None)
def _agg1_kernel():
    return pl.kernel(
        _agg1_body,
        out_type=jax.ShapeDtypeStruct((NCHUNK, N_PAD, CW), jnp.float32),
        mesh=_mesh(),
        compiler_params=_SC_PARAMS,
        scratch_types=(
            [
                pltpu.VMEM((ROWS_ALL // NS, BATCH), jnp.int32),
                pltpu.VMEM((ROWS_ALL // NS, BATCH), jnp.int32),
                pltpu.VMEM((ROWS_ALL // NS, BATCH), jnp.int32),
            ]
            + [pltpu.VMEM((BATCH, CW), jnp.float32)] * (NBUF + 1)
            + [pltpu.VMEM_SHARED((N_PAD, CW), jnp.float32)]
            + [pltpu.SemaphoreType.DMA] * NBUF
        ),
    )


def _agg2_body(gs_hbm, src2d_hbm, dst2d_hbm, out_hbm,
               srcv, dstv, b0, b1, b2, b3, zbuf, acc_sh, s0, s1, s2, s3):
    cid = lax.axis_index("c")
    sid = lax.axis_index("s")
    wid = cid * NS + sid
    rows_per_tile = ROWS_ALL // NW
    bufs = [b0, b1, b2, b3]
    sems = [s0, s1, s2, s3]

    pltpu.sync_copy(src2d_hbm.at[pl.ds(wid * rows_per_tile, rows_per_tile)],
                    srcv)
    pltpu.sync_copy(dst2d_hbm.at[pl.ds(wid * rows_per_tile, rows_per_tile)],
                    dstv)

    _zero_vmem(zbuf, BATCH, D2)

    def zacc(i, _):
        pltpu.sync_copy(zbuf, acc_sh.at[pl.ds(sid * NT + i * BATCH, BATCH)])
        return 0
    lax.fori_loop(0, NT // BATCH, zacc, 0)
    plsc.subcore_barrier()

    _ring_agg(gs_hbm, srcv, dstv, acc_sh, bufs, sems, rows_per_tile)
    plsc.subcore_barrier()

    def flush(i, _):
        r0 = sid * NT + i * BATCH
        pltpu.sync_copy(acc_sh.at[pl.ds(r0, BATCH)], b0)
        pltpu.sync_copy(b0, out_hbm.at[cid, pl.ds(r0, BATCH)])
        return 0
    lax.fori_loop(0, NT // BATCH, flush, 0)


@functools.lru_cache(maxsize=None)
def _agg2_kernel():
    return pl.kernel(
        _agg2_body,
        out_type=jax.ShapeDtypeStruct((NC, N_PAD, D2), jnp.float32),
        mesh=_mesh(),
        compiler_params=_SC_PARAMS,
        scratch_types=(
            [
                pltpu.VMEM((ROWS_ALL // NW, BATCH), jnp.int32),
                pltpu.VMEM((ROWS_ALL // NW, BATCH), jnp.int32),
            ]
            + [pltpu.VMEM((BATCH, D2), jnp.float32)] * (NBUF + 1)
            + [pltpu.VMEM_SHARED((N_PAD, D2), jnp.float32)]
            + [pltpu.SemaphoreType.DMA] * NBUF
        ),
    )


def _pre_body(degp_ref, x_ref, xcat_ref, dinv_ref):
    deg = jnp.sum(degp_ref[...], axis=1, keepdims=True)
    dcol = lax.rsqrt(deg + 1.0)
    dinv_ref[...] = dcol
    xs = x_ref[...] * dcol
    rb = x_ref.shape[0]
    for c in range(NCHUNK):
        lo = c * CW
        if lo + CW <= 600:
            xcat_ref[c] = xs[:, lo:lo + CW]
        else:
            xcat_ref[c] = jnp.concatenate(
                [xs[:, lo:600], jnp.zeros((rb, lo + CW - 600), jnp.float32)],
                axis=1)


def _pre(degp, x):
    rb = 2000
    return pl.pallas_call(
        _pre_body,
        grid=(N // rb,),
        in_specs=[
            pl.BlockSpec((rb, NW), lambda i: (i, 0)),
            pl.BlockSpec((rb, 600), lambda i: (i, 0)),
        ],
        out_specs=[
            pl.BlockSpec((NCHUNK, rb, CW), lambda i: (0, i, 0)),
            pl.BlockSpec((rb, 1), lambda i: (i, 0)),
        ],
        out_shape=[
            jax.ShapeDtypeStruct((NCHUNK, N, CW), jnp.float32),
            jax.ShapeDtypeStruct((N, 1), jnp.float32),
        ],
    )(degp, x)


def _mid_body(agg_ref, xcat_ref, dinv_ref, w1_ref, b1_ref, w2_ref, gs_ref):
    dcol = dinv_ref[...]
    agg = jnp.concatenate([agg_ref[c] for c in range(NCHUNK)], axis=1)
    xs = jnp.concatenate([xcat_ref[c] for c in range(NCHUNK)], axis=1)
    u = (agg[:, :600] + xs[:, :600]) * dcol
    h = jax.lax.dot_general(u, w1_ref[...], (((1,), (1,)), ((), ())),
                            preferred_element_type=jnp.float32)
    h = jnp.maximum(h + b1_ref[...], 0.0)
    g = jax.lax.dot_general(h, w2_ref[...], (((1,), (1,)), ((), ())),
                            preferred_element_type=jnp.float32)
    gs_ref[...] = g * dcol


def _mid(agg8, xcat8, dinv, w1, b1, w2):
    rb = 2000
    return pl.pallas_call(
        _mid_body,
        grid=(N // rb,),
        in_specs=[
            pl.BlockSpec((NCHUNK, rb, CW), lambda i: (0, i, 0)),
            pl.BlockSpec((NCHUNK, rb, CW), lambda i: (0, i, 0)),
            pl.BlockSpec((rb, 1), lambda i: (i, 0)),
            pl.BlockSpec((628, 600), lambda i: (0, 0)),
            pl.BlockSpec((628,), lambda i: (0,)),
            pl.BlockSpec((D2, 628), lambda i: (0, 0)),
        ],
        out_specs=pl.BlockSpec((rb, D2), lambda i: (i, 0)),
        out_shape=jax.ShapeDtypeStruct((N, D2), jnp.float32),
    )(agg8, xcat8, dinv, w1, b1, w2)


def _fin_body(p0_ref, p1_ref, gs_ref, dinv_ref, b2_ref, z_ref):
    dcol = dinv_ref[...]
    z_ref[...] = ((p0_ref[0] + p1_ref[0] + gs_ref[...]) * dcol
                  + b2_ref[...])


def _fin(parts, gs, dinv, b2):
    rb = 2000
    return pl.pallas_call(
        _fin_body,
        grid=(N // rb,),
        in_specs=[
            pl.BlockSpec((1, rb, D2), lambda i: (0, i, 0)),
            pl.BlockSpec((1, rb, D2), lambda i: (1, i, 0)),
            pl.BlockSpec((rb, D2), lambda i: (i, 0)),
            pl.BlockSpec((rb, 1), lambda i: (i, 0)),
            pl.BlockSpec((D2,), lambda i: (0,)),
        ],
        out_specs=pl.BlockSpec((rb, D2), lambda i: (i, 0)),
        out_shape=jax.ShapeDtypeStruct((N, D2), jnp.float32),
    )(parts, parts, gs, dinv, b2)


@jax.jit
def kernel(x, pos_edge_index, W1, b1, W2, b2):
    src = pos_edge_index[0].astype(jnp.int32)
    dst = pos_edge_index[1].astype(jnp.int32)
    src2d = jnp.concatenate([src, jnp.zeros((E_PAD - E,), jnp.int32)])
    src2d = src2d.reshape(ROWS_ALL, BATCH)
    dst2d = jnp.concatenate([dst, jnp.full((E_PAD - E,), N, jnp.int32)])
    dst2d = dst2d.reshape(ROWS_ALL, BATCH)

    degp = _deg_kernel()(dst2d)
    degp_t = degp[:, :N].T
    xcat8, dinv = _pre(degp_t, x)
    xs_cat = xcat8.reshape(NCHUNK * N, CW)

    agg8 = _agg1_kernel()(xs_cat, src2d, dst2d)
    gs = _mid(agg8, xcat8, dinv, W1, b1, W2)
    parts = _agg2_kernel()(gs, src2d, dst2d)
    return _fin(parts, gs, dinv, b2)
